# baseline (speedup 1.0000x reference)
import functools

import jax
import jax.numpy as jnp
from jax import lax
from jax.experimental import pallas as pl
from jax.experimental.pallas import tpu as pltpu
from jax.experimental.pallas import tpu_sc as plsc

_C = 1.0
_MIN_NORM = 1e-15
_BALL_EPS = 4e-3
_MAXNORM = (1.0 - _BALL_EPS)

_NC = 2
_NS = 16
_NW = _NC * _NS
_ECHUNK = 128


def _artanh(x):
    x = jnp.clip(x, -1.0 + 1e-7, 1.0 - 1e-7)
    return 0.5 * (jnp.log1p(x) - jnp.log1p(-x))


def _rnorm(x):
    return jnp.clip(
        jnp.sqrt(jnp.sum(x * x, axis=-1, keepdims=True)), _MIN_NORM, None
    )


def _proj(x):
    norm = _rnorm(x)
    return jnp.where(norm > _MAXNORM, x / norm * _MAXNORM, x)


def _expmap0(u):
    u_norm = _rnorm(u)
    return jnp.tanh(u_norm) * u / u_norm


def _logmap0(p):
    p_norm = _rnorm(p)
    return _artanh(p_norm) / p_norm * p


def _mobius_add(x, y):
    x2 = jnp.sum(x * x, axis=-1, keepdims=True)
    y2 = jnp.sum(y * y, axis=-1, keepdims=True)
    xy = jnp.sum(x * y, axis=-1, keepdims=True)
    num = (1.0 + 2.0 * xy + y2) * x + (1.0 - x2) * y
    denom = 1.0 + 2.0 * xy + x2 * y2
    return num / jnp.clip(denom, _MIN_NORM, None)


def _pre_body(x_ref, w_ref, b_ref, out_ref):
    x = x_ref[...]
    w = w_ref[...]
    b = b_ref[...]
    u = _rnorm(x)
    m0 = lax.dot_general(
        x, w, (((1,), (1,)), ((), ())),
        preferred_element_type=jnp.float32,
    )
    mu = _rnorm(m0)
    a = jnp.minimum(jnp.tanh(u), _MAXNORM)
    r = jnp.tanh(mu / u * _artanh(a))
    rho = jnp.minimum(r, _MAXNORM)
    cond = jnp.all(m0 == 0.0, axis=-1, keepdims=True)
    rho = jnp.where(cond, 0.0, rho)
    res = (rho / mu) * m0
    hyp_b = _proj(_expmap0(b))
    x2 = rho * rho
    y2 = jnp.sum(hyp_b * hyp_b, axis=-1, keepdims=True)
    xy = jnp.sum(res * hyp_b, axis=-1, keepdims=True)
    num = (1.0 + 2.0 * xy + y2) * res + (1.0 - x2) * hyp_b
    denom = 1.0 + 2.0 * xy + x2 * y2
    h = num / jnp.clip(denom, _MIN_NORM, None)
    hn = _rnorm(h)
    pn = jnp.minimum(hn, _MAXNORM)
    f = _artanh(pn) / pn * jnp.where(hn > _MAXNORM, _MAXNORM / hn, 1.0)
    out_ref[...] = f * h


def _stage_pre(x, w, b2d, blk):
    n, d = x.shape
    grid = n // blk
    return pl.pallas_call(
        _pre_body,
        grid=(grid,),
        in_specs=[
            pl.BlockSpec((blk, d), lambda i: (i, 0)),
            pl.BlockSpec((d, d), lambda i: (0, 0)),
            pl.BlockSpec((1, d), lambda i: (0, 0)),
        ],
        out_specs=pl.BlockSpec((blk, d), lambda i: (i, 0)),
        out_shape=jax.ShapeDtypeStruct((n, d), jnp.float32),
    )(x, w, b2d)


def _make_seg_sum(n_acc, d, rows_per_w):
    acc_per_s = n_acc // _NS
    n_pairs = rows_per_w // 2
    mesh = plsc.VectorSubcoreMesh(core_axis_name="c", subcore_axis_name="s")

    @functools.partial(
        pl.kernel,
        out_type=jax.ShapeDtypeStruct((_NC, n_acc, d), jnp.float32),
        mesh=mesh,
        scratch_types=[
            pltpu.VMEM((rows_per_w, _ECHUNK), jnp.int32),
            pltpu.VMEM((_ECHUNK,), jnp.int32),
            pltpu.VMEM((_ECHUNK,), jnp.int32),
            pltpu.VMEM((_ECHUNK, d), jnp.float32),
            pltpu.VMEM((_ECHUNK, d), jnp.float32),
            pltpu.VMEM_SHARED((n_acc, d), jnp.float32),
            pltpu.SemaphoreType.DMA,
            pltpu.SemaphoreType.DMA,
            pltpu.SemaphoreType.DMA,
            pltpu.SemaphoreType.DMA,
            pltpu.SemaphoreType.DMA,
            pltpu.SemaphoreType.DMA,
            pltpu.SemaphoreType.DMA,
        ],
    )
    def seg_sum(xt_hbm, src_hbm, dst_hbm, zeros_hbm, out_hbm,
                src_v, dst_v0, dst_v1, rows0, rows1, acc,
                isem, dsem0, dsem1, gsem0, gsem1, ssem0, ssem1):
        cid = lax.axis_index("c")
        sid = lax.axis_index("s")
        wid = cid * _NS + sid
        base = rows_per_w * wid
        icp = pltpu.async_copy(
            src_hbm.at[pl.ds(base, rows_per_w)], src_v, isem)
        pltpu.sync_copy(zeros_hbm, acc.at[pl.ds(sid * acc_per_s, acc_per_s)])
        icp.wait()
        plsc.subcore_barrier()

        pltpu.async_copy(dst_hbm.at[base], dst_v0, dsem0)
        pltpu.async_copy(xt_hbm.at[src_v.at[0]], rows0, gsem0)

        def body(j, carry):
            i = 2 * j
            pltpu.make_async_copy(xt_hbm.at[src_v.at[i]], rows0, gsem0).wait()
            pltpu.make_async_copy(dst_hbm.at[base + i], dst_v0, dsem0).wait()
            pltpu.async_copy(rows0, acc.at[dst_v0], ssem0, add=True)

            @pl.when(j > 0)
            def _():
                pltpu.make_async_copy(rows1, acc.at[dst_v1], ssem1).wait()

            pltpu.async_copy(dst_hbm.at[base + i + 1], dst_v1, dsem1)
            pltpu.async_copy(xt_hbm.at[src_v.at[i + 1]], rows1, gsem1)
            pltpu.make_async_copy(
                xt_hbm.at[src_v.at[i + 1]], rows1, gsem1).wait()
            pltpu.make_async_copy(
                dst_hbm.at[base + i + 1], dst_v1, dsem1).wait()
            pltpu.async_copy(rows1, acc.at[dst_v1], ssem1, add=True)

            @pl.when(j + 1 < n_pairs)
            def _():
                pltpu.make_async_copy(rows0, acc.at[dst_v0], ssem0).wait()
                pltpu.async_copy(dst_hbm.at[base + i + 2], dst_v0, dsem0)
                pltpu.async_copy(xt_hbm.at[src_v.at[i + 2]], rows0, gsem0)

            return carry

        lax.fori_loop(0, n_pairs, body, 0)
        pltpu.make_async_copy(rows0, acc.at[dst_v0], ssem0).wait()
        pltpu.make_async_copy(rows1, acc.at[dst_v1], ssem1).wait()
        plsc.subcore_barrier()
        pltpu.sync_copy(
            acc.at[pl.ds(sid * acc_per_s, acc_per_s)],
            out_hbm.at[cid, pl.ds(sid * acc_per_s, acc_per_s)],
        )

    return seg_sum


def _post_body(p_ref, out_ref):
    s = p_ref[0] + p_ref[1]
    sn = _rnorm(s)
    a1 = jnp.minimum(jnp.tanh(sn), _MAXNORM)
    kappa = _artanh(a1) / sn
    rs = jax.nn.relu(s)
    rn = jnp.sqrt(jnp.sum(rs * rs, axis=-1, keepdims=True))
    xn = jnp.clip(kappa * rn, _MIN_NORM, None)
    a2 = jnp.minimum(jnp.tanh(xn), _MAXNORM)
    out_ref[...] = (a2 / xn * kappa) * rs


def _stage_post(partials, n, blk):
    nc, n_acc, d = partials.shape
    grid = n // blk
    return pl.pallas_call(
        _post_body,
        grid=(grid,),
        in_specs=[pl.BlockSpec((nc, blk, d), lambda i: (0, i, 0))],
        out_specs=pl.BlockSpec((blk, d), lambda i: (i, 0)),
        out_shape=jax.ShapeDtypeStruct((n, d), jnp.float32),
    )(partials)


def kernel(x, edge_index, W, b):
    n, d = x.shape
    e = edge_index.shape[1]
    src = edge_index[0]
    dst = edge_index[1]

    grain = _NW * _ECHUNK * 2
    e_pad = ((e + grain - 1) // grain) * grain
    rows_per_w = e_pad // (_NW * _ECHUNK)
    _ALIGN = _NS * 8
    n_acc = ((n + _ALIGN - 1) // _ALIGN) * _ALIGN
    if e_pad != e:
        pad = e_pad - e
        ar = jnp.arange(pad, dtype=jnp.int32)
        pad_src = ar % n
        pad_dst = n_acc + (ar % 128)
        n_acc += 128
        src = jnp.concatenate([src, pad_src])
        dst = jnp.concatenate([dst, pad_dst])
    src2 = src.reshape(e_pad // _ECHUNK, _ECHUNK)
    dst2 = dst.reshape(e_pad // _ECHUNK, _ECHUNK)

    x_t = _stage_pre(x, W, b.reshape(1, -1), blk=1000)

    zeros = jnp.zeros((n_acc // _NS, d), jnp.float32)
    seg = _make_seg_sum(n_acc, d, rows_per_w)
    partials = seg(x_t, src2, dst2, zeros)

    return _stage_post(partials, n, blk=1000)


# speedup vs baseline: 1.1498x; 1.1498x over previous
import functools

import jax
import jax.numpy as jnp
from jax import lax
from jax.experimental import pallas as pl
from jax.experimental.pallas import tpu as pltpu
from jax.experimental.pallas import tpu_sc as plsc

_C = 1.0
_MIN_NORM = 1e-15
_BALL_EPS = 4e-3
_MAXNORM = (1.0 - _BALL_EPS)

_NC = 2
_NS = 16
_NW = _NC * _NS
_ECHUNK = 128


def _artanh(x):
    x = jnp.clip(x, -1.0 + 1e-7, 1.0 - 1e-7)
    return 0.5 * (jnp.log1p(x) - jnp.log1p(-x))


def _rnorm(x):
    return jnp.clip(
        jnp.sqrt(jnp.sum(x * x, axis=-1, keepdims=True)), _MIN_NORM, None
    )


def _proj(x):
    norm = _rnorm(x)
    return jnp.where(norm > _MAXNORM, x / norm * _MAXNORM, x)


def _expmap0(u):
    u_norm = _rnorm(u)
    return jnp.tanh(u_norm) * u / u_norm


def _logmap0(p):
    p_norm = _rnorm(p)
    return _artanh(p_norm) / p_norm * p


def _mobius_add(x, y):
    x2 = jnp.sum(x * x, axis=-1, keepdims=True)
    y2 = jnp.sum(y * y, axis=-1, keepdims=True)
    xy = jnp.sum(x * y, axis=-1, keepdims=True)
    num = (1.0 + 2.0 * xy + y2) * x + (1.0 - x2) * y
    denom = 1.0 + 2.0 * xy + x2 * y2
    return num / jnp.clip(denom, _MIN_NORM, None)


def _pre_body(x_ref, w_ref, b_ref, out_ref):
    x = x_ref[...]
    w = w_ref[...]
    b = b_ref[...]
    u = _rnorm(x)
    m0 = lax.dot_general(
        x, w, (((1,), (1,)), ((), ())),
        preferred_element_type=jnp.float32,
    )
    mu2 = jnp.sum(m0 * m0, axis=-1, keepdims=True)
    mu = jnp.clip(jnp.sqrt(mu2), _MIN_NORM, None)
    a = jnp.minimum(jnp.tanh(u), _MAXNORM)
    r = jnp.tanh(mu / u * _artanh(a))
    rho = jnp.minimum(r, _MAXNORM)
    rho = jnp.where(mu2 == 0.0, 0.0, rho)
    hyp_b = _proj(_expmap0(b))
    g = rho / mu
    x2 = rho * rho
    y2 = jnp.sum(hyp_b * hyp_b, axis=-1, keepdims=True)
    xy = g * lax.dot_general(
        m0, hyp_b, (((1,), (1,)), ((), ())),
        preferred_element_type=jnp.float32,
    )
    num = ((1.0 + 2.0 * xy + y2) * g) * m0 + (1.0 - x2) * hyp_b
    denom = 1.0 + 2.0 * xy + x2 * y2
    h = num / jnp.clip(denom, _MIN_NORM, None)
    hn = _rnorm(h)
    pn = jnp.minimum(hn, _MAXNORM)
    f = _artanh(pn) / pn * jnp.where(hn > _MAXNORM, _MAXNORM / hn, 1.0)
    out_ref[...] = f * h


def _stage_pre(x, w, b2d, blk):
    n, d = x.shape
    grid = n // blk
    return pl.pallas_call(
        _pre_body,
        grid=(grid,),
        in_specs=[
            pl.BlockSpec((blk, d), lambda i: (i, 0)),
            pl.BlockSpec((d, d), lambda i: (0, 0)),
            pl.BlockSpec((1, d), lambda i: (0, 0)),
        ],
        out_specs=pl.BlockSpec((blk, d), lambda i: (i, 0)),
        out_shape=jax.ShapeDtypeStruct((n, d), jnp.float32),
    )(x, w, b2d)


def _make_seg_sum(n_acc, d, rows_per_w):
    acc_per_s = n_acc // _NS
    n_pairs = rows_per_w // 2
    mesh = plsc.VectorSubcoreMesh(core_axis_name="c", subcore_axis_name="s")

    @functools.partial(
        pl.kernel,
        out_type=jax.ShapeDtypeStruct((_NC, n_acc, d), jnp.float32),
        mesh=mesh,
        scratch_types=[
            pltpu.VMEM((rows_per_w, _ECHUNK), jnp.int32),
            pltpu.VMEM((_ECHUNK,), jnp.int32),
            pltpu.VMEM((_ECHUNK,), jnp.int32),
            pltpu.VMEM((_ECHUNK, d), jnp.float32),
            pltpu.VMEM((_ECHUNK, d), jnp.float32),
            pltpu.VMEM_SHARED((n_acc, d), jnp.float32),
            pltpu.SemaphoreType.DMA,
            pltpu.SemaphoreType.DMA,
            pltpu.SemaphoreType.DMA,
            pltpu.SemaphoreType.DMA,
            pltpu.SemaphoreType.DMA,
        ],
    )
    def seg_sum(xt_hbm, src_hbm, dst_hbm, zeros_hbm, out_hbm,
                src_v, dst_v0, dst_v1, rows0, rows1, acc,
                isem, dsem0, dsem1, gsem0, gsem1):
        cid = lax.axis_index("c")
        sid = lax.axis_index("s")
        wid = cid * _NS + sid
        base = rows_per_w * wid
        icp = pltpu.async_copy(
            src_hbm.at[pl.ds(base, rows_per_w)], src_v, isem)
        pltpu.sync_copy(zeros_hbm, acc.at[pl.ds(sid * acc_per_s, acc_per_s)])
        icp.wait()
        plsc.subcore_barrier()

        pltpu.async_copy(dst_hbm.at[base], dst_v0, dsem0)
        pltpu.async_copy(xt_hbm.at[src_v.at[0]], rows0, gsem0)

        def body(j, carry):
            i = 2 * j
            pltpu.async_copy(dst_hbm.at[base + i + 1], dst_v1, dsem1)
            pltpu.async_copy(xt_hbm.at[src_v.at[i + 1]], rows1, gsem1)
            pltpu.make_async_copy(xt_hbm.at[src_v.at[i]], rows0, gsem0).wait()
            pltpu.make_async_copy(dst_hbm.at[base + i], dst_v0, dsem0).wait()
            pltpu.sync_copy(rows0, acc.at[dst_v0], add=True)

            @pl.when(j + 1 < n_pairs)
            def _():
                pltpu.async_copy(dst_hbm.at[base + i + 2], dst_v0, dsem0)
                pltpu.async_copy(xt_hbm.at[src_v.at[i + 2]], rows0, gsem0)

            pltpu.make_async_copy(
                xt_hbm.at[src_v.at[i + 1]], rows1, gsem1).wait()
            pltpu.make_async_copy(
                dst_hbm.at[base + i + 1], dst_v1, dsem1).wait()
            pltpu.sync_copy(rows1, acc.at[dst_v1], add=True)
            return carry

        lax.fori_loop(0, n_pairs, body, 0)
        plsc.subcore_barrier()
        pltpu.sync_copy(
            acc.at[pl.ds(sid * acc_per_s, acc_per_s)],
            out_hbm.at[cid, pl.ds(sid * acc_per_s, acc_per_s)],
        )

    return seg_sum


def _post_body(p_ref, out_ref):
    s = p_ref[0] + p_ref[1]
    rs = jax.nn.relu(s)
    sn = _rnorm(s)
    rn = jnp.sqrt(jnp.sum(rs * rs, axis=-1, keepdims=True))
    a1 = jnp.minimum(jnp.tanh(sn), _MAXNORM)
    kappa = _artanh(a1) / sn
    xn = jnp.clip(kappa * rn, _MIN_NORM, None)
    a2 = jnp.minimum(jnp.tanh(xn), _MAXNORM)
    out_ref[...] = (a2 / xn * kappa) * rs


def _stage_post(partials, n, blk):
    nc, n_acc, d = partials.shape
    grid = n // blk
    return pl.pallas_call(
        _post_body,
        grid=(grid,),
        in_specs=[pl.BlockSpec((nc, blk, d), lambda i: (0, i, 0))],
        out_specs=pl.BlockSpec((blk, d), lambda i: (i, 0)),
        out_shape=jax.ShapeDtypeStruct((n, d), jnp.float32),
    )(partials)


def kernel(x, edge_index, W, b):
    n, d = x.shape
    e = edge_index.shape[1]
    src = edge_index[0]
    dst = edge_index[1]

    grain = _NW * _ECHUNK * 2
    e_pad = ((e + grain - 1) // grain) * grain
    rows_per_w = e_pad // (_NW * _ECHUNK)
    _ALIGN = _NS * 8
    n_acc = ((n + _ALIGN - 1) // _ALIGN) * _ALIGN
    if e_pad != e:
        pad = e_pad - e
        ar = jnp.arange(pad, dtype=jnp.int32)
        pad_src = ar % n
        pad_dst = n_acc + (ar % 128)
        n_acc += 128
        src = jnp.concatenate([src, pad_src])
        dst = jnp.concatenate([dst, pad_dst])
    src2 = src.reshape(e_pad // _ECHUNK, _ECHUNK)
    dst2 = dst.reshape(e_pad // _ECHUNK, _ECHUNK)

    x_t = _stage_pre(x, W, b.reshape(1, -1), blk=2000)

    zeros = jnp.zeros((n_acc // _NS, d), jnp.float32)
    seg = _make_seg_sum(n_acc, d, rows_per_w)
    partials = seg(x_t, src2, dst2, zeros)

    return _stage_post(partials, n, blk=2000)


# speedup vs baseline: 1.1862x; 1.0316x over previous
import functools

import jax
import jax.numpy as jnp
from jax import lax
from jax.experimental import pallas as pl
from jax.experimental.pallas import tpu as pltpu
from jax.experimental.pallas import tpu_sc as plsc

_C = 1.0
_MIN_NORM = 1e-15
_BALL_EPS = 4e-3
_MAXNORM = (1.0 - _BALL_EPS)
_ATANH_MAXNORM = 3.1063671188198696

_NC = 2
_NS = 16
_NW = _NC * _NS
_ECHUNK = 128


def _artanh(x):
    x = jnp.clip(x, -1.0 + 1e-7, 1.0 - 1e-7)
    return 0.5 * (jnp.log1p(x) - jnp.log1p(-x))


def _rnorm(x):
    return jnp.clip(
        jnp.sqrt(jnp.sum(x * x, axis=-1, keepdims=True)), _MIN_NORM, None
    )


def _proj(x):
    norm = _rnorm(x)
    return jnp.where(norm > _MAXNORM, x / norm * _MAXNORM, x)


def _expmap0(u):
    u_norm = _rnorm(u)
    return jnp.tanh(u_norm) * u / u_norm


def _logmap0(p):
    p_norm = _rnorm(p)
    return _artanh(p_norm) / p_norm * p


def _mobius_add(x, y):
    x2 = jnp.sum(x * x, axis=-1, keepdims=True)
    y2 = jnp.sum(y * y, axis=-1, keepdims=True)
    xy = jnp.sum(x * y, axis=-1, keepdims=True)
    num = (1.0 + 2.0 * xy + y2) * x + (1.0 - x2) * y
    denom = 1.0 + 2.0 * xy + x2 * y2
    return num / jnp.clip(denom, _MIN_NORM, None)


def _pre_body(x_ref, w_ref, b_ref, out_ref):
    x = x_ref[...]
    w = w_ref[...]
    b = b_ref[...]
    u = _rnorm(x)
    m0 = lax.dot_general(
        x, w, (((1,), (1,)), ((), ())),
        preferred_element_type=jnp.float32,
    )
    mu2 = jnp.sum(m0 * m0, axis=-1, keepdims=True)
    mu = jnp.clip(jnp.sqrt(mu2), _MIN_NORM, None)
    r = jnp.tanh(mu / u * jnp.minimum(u, _ATANH_MAXNORM))
    rho = jnp.minimum(r, _MAXNORM)
    rho = jnp.where(mu2 == 0.0, 0.0, rho)
    hyp_b = _proj(_expmap0(b))
    g = rho / mu
    x2 = rho * rho
    y2 = jnp.sum(hyp_b * hyp_b, axis=-1, keepdims=True)
    xy = g * lax.dot_general(
        m0, hyp_b, (((1,), (1,)), ((), ())),
        preferred_element_type=jnp.float32,
    )
    num = ((1.0 + 2.0 * xy + y2) * g) * m0 + (1.0 - x2) * hyp_b
    denom = 1.0 + 2.0 * xy + x2 * y2
    h = num / jnp.clip(denom, _MIN_NORM, None)
    hn = _rnorm(h)
    pn = jnp.minimum(hn, _MAXNORM)
    f = _artanh(pn) / pn * jnp.where(hn > _MAXNORM, _MAXNORM / hn, 1.0)
    out_ref[...] = f * h


def _stage_pre(x, w, b2d, blk):
    n, d = x.shape
    grid = n // blk
    return pl.pallas_call(
        _pre_body,
        grid=(grid,),
        in_specs=[
            pl.BlockSpec((blk, d), lambda i: (i, 0)),
            pl.BlockSpec((d, d), lambda i: (0, 0)),
            pl.BlockSpec((1, d), lambda i: (0, 0)),
        ],
        out_specs=pl.BlockSpec((blk, d), lambda i: (i, 0)),
        out_shape=jax.ShapeDtypeStruct((n, d), jnp.float32),
    )(x, w, b2d)


def _make_seg_sum(n_acc, d, rows_per_w):
    acc_per_s = n_acc // _NS
    n_pairs = rows_per_w // 2
    mesh = plsc.VectorSubcoreMesh(core_axis_name="c", subcore_axis_name="s")

    @functools.partial(
        pl.kernel,
        out_type=jax.ShapeDtypeStruct((_NC, n_acc, d), jnp.float32),
        mesh=mesh,
        scratch_types=[
            pltpu.VMEM((rows_per_w, _ECHUNK), jnp.int32),
            pltpu.VMEM((_ECHUNK,), jnp.int32),
            pltpu.VMEM((_ECHUNK,), jnp.int32),
            pltpu.VMEM((_ECHUNK, d), jnp.float32),
            pltpu.VMEM((_ECHUNK, d), jnp.float32),
            pltpu.VMEM_SHARED((n_acc, d), jnp.float32),
            pltpu.SemaphoreType.DMA,
            pltpu.SemaphoreType.DMA,
            pltpu.SemaphoreType.DMA,
            pltpu.SemaphoreType.DMA,
            pltpu.SemaphoreType.DMA,
        ],
    )
    def seg_sum(xt_hbm, src_hbm, dst_hbm, zeros_hbm, out_hbm,
                src_v, dst_v0, dst_v1, rows0, rows1, acc,
                isem, dsem0, dsem1, gsem0, gsem1):
        cid = lax.axis_index("c")
        sid = lax.axis_index("s")
        wid = cid * _NS + sid
        base = rows_per_w * wid
        icp = pltpu.async_copy(
            src_hbm.at[pl.ds(base, rows_per_w)], src_v, isem)
        pltpu.sync_copy(zeros_hbm, acc.at[pl.ds(sid * acc_per_s, acc_per_s)])
        icp.wait()
        plsc.subcore_barrier()

        pltpu.async_copy(dst_hbm.at[base], dst_v0, dsem0)
        pltpu.async_copy(xt_hbm.at[src_v.at[0]], rows0, gsem0)

        def body(j, carry):
            i = 2 * j
            pltpu.async_copy(dst_hbm.at[base + i + 1], dst_v1, dsem1)
            pltpu.async_copy(xt_hbm.at[src_v.at[i + 1]], rows1, gsem1)
            pltpu.make_async_copy(xt_hbm.at[src_v.at[i]], rows0, gsem0).wait()
            pltpu.make_async_copy(dst_hbm.at[base + i], dst_v0, dsem0).wait()
            pltpu.sync_copy(rows0, acc.at[dst_v0], add=True)

            @pl.when(j + 1 < n_pairs)
            def _():
                pltpu.async_copy(dst_hbm.at[base + i + 2], dst_v0, dsem0)
                pltpu.async_copy(xt_hbm.at[src_v.at[i + 2]], rows0, gsem0)

            pltpu.make_async_copy(
                xt_hbm.at[src_v.at[i + 1]], rows1, gsem1).wait()
            pltpu.make_async_copy(
                dst_hbm.at[base + i + 1], dst_v1, dsem1).wait()
            pltpu.sync_copy(rows1, acc.at[dst_v1], add=True)
            return carry

        lax.fori_loop(0, n_pairs, body, 0)
        plsc.subcore_barrier()
        pltpu.sync_copy(
            acc.at[pl.ds(sid * acc_per_s, acc_per_s)],
            out_hbm.at[cid, pl.ds(sid * acc_per_s, acc_per_s)],
        )

    return seg_sum


def _post_body(p_ref, out_ref):
    s = p_ref[0] + p_ref[1]
    rs = jax.nn.relu(s)
    sn = _rnorm(s)
    rn = jnp.sqrt(jnp.sum(rs * rs, axis=-1, keepdims=True))
    kappa = jnp.minimum(sn, _ATANH_MAXNORM) / sn
    xn = jnp.clip(kappa * rn, _MIN_NORM, None)
    a2 = jnp.minimum(jnp.tanh(xn), _MAXNORM)
    out_ref[...] = (a2 / xn * kappa) * rs


def _stage_post(partials, n, blk):
    nc, n_acc, d = partials.shape
    grid = n // blk
    return pl.pallas_call(
        _post_body,
        grid=(grid,),
        in_specs=[pl.BlockSpec((nc, blk, d), lambda i: (0, i, 0))],
        out_specs=pl.BlockSpec((blk, d), lambda i: (i, 0)),
        out_shape=jax.ShapeDtypeStruct((n, d), jnp.float32),
    )(partials)


def kernel(x, edge_index, W, b):
    n, d = x.shape
    e = edge_index.shape[1]
    src = edge_index[0]
    dst = edge_index[1]

    grain = _NW * _ECHUNK * 2
    e_pad = ((e + grain - 1) // grain) * grain
    rows_per_w = e_pad // (_NW * _ECHUNK)
    _ALIGN = _NS * 8
    n_acc = ((n + _ALIGN - 1) // _ALIGN) * _ALIGN
    if e_pad != e:
        pad = e_pad - e
        ar = jnp.arange(pad, dtype=jnp.int32)
        pad_src = ar % n
        pad_dst = n_acc + (ar % 128)
        n_acc += 128
        src = jnp.concatenate([src, pad_src])
        dst = jnp.concatenate([dst, pad_dst])
    src2 = src.reshape(e_pad // _ECHUNK, _ECHUNK)
    dst2 = dst.reshape(e_pad // _ECHUNK, _ECHUNK)

    x_t = _stage_pre(x, W, b.reshape(1, -1), blk=2000)

    zeros = jnp.zeros((n_acc // _NS, d), jnp.float32)
    seg = _make_seg_sum(n_acc, d, rows_per_w)
    partials = seg(x_t, src2, dst2, zeros)

    return _stage_post(partials, n, blk=2000)


# speedup vs baseline: 1.1890x; 1.0024x over previous
import functools

import jax
import jax.numpy as jnp
from jax import lax
from jax.experimental import pallas as pl
from jax.experimental.pallas import tpu as pltpu
from jax.experimental.pallas import tpu_sc as plsc

_C = 1.0
_MIN_NORM = 1e-15
_BALL_EPS = 4e-3
_MAXNORM = (1.0 - _BALL_EPS)
_ATANH_MAXNORM = 3.1063671188198696

_NC = 2
_NS = 16
_NW = _NC * _NS
_ECHUNK = 128


def _artanh(x):
    x = jnp.clip(x, -1.0 + 1e-7, 1.0 - 1e-7)
    return 0.5 * (jnp.log1p(x) - jnp.log1p(-x))


def _rnorm(x):
    return jnp.clip(
        jnp.sqrt(jnp.sum(x * x, axis=-1, keepdims=True)), _MIN_NORM, None
    )


def _proj(x):
    norm = _rnorm(x)
    return jnp.where(norm > _MAXNORM, x / norm * _MAXNORM, x)


def _expmap0(u):
    u_norm = _rnorm(u)
    return jnp.tanh(u_norm) * u / u_norm


def _logmap0(p):
    p_norm = _rnorm(p)
    return _artanh(p_norm) / p_norm * p


def _mobius_add(x, y):
    x2 = jnp.sum(x * x, axis=-1, keepdims=True)
    y2 = jnp.sum(y * y, axis=-1, keepdims=True)
    xy = jnp.sum(x * y, axis=-1, keepdims=True)
    num = (1.0 + 2.0 * xy + y2) * x + (1.0 - x2) * y
    denom = 1.0 + 2.0 * xy + x2 * y2
    return num / jnp.clip(denom, _MIN_NORM, None)


def _pre_body(x_ref, w_ref, b_ref, out_ref):
    x = x_ref[...]
    w = w_ref[...]
    b = b_ref[...]
    u = _rnorm(x)
    m0 = lax.dot_general(
        x, w, (((1,), (1,)), ((), ())),
        preferred_element_type=jnp.float32,
    )
    mu2 = jnp.sum(m0 * m0, axis=-1, keepdims=True)
    mu = jnp.clip(jnp.sqrt(mu2), _MIN_NORM, None)
    r = jnp.tanh(mu / u * jnp.minimum(u, _ATANH_MAXNORM))
    rho = jnp.minimum(r, _MAXNORM)
    rho = jnp.where(mu2 == 0.0, 0.0, rho)
    hyp_b = _proj(_expmap0(b))
    g = rho / mu
    x2 = rho * rho
    y2 = jnp.sum(hyp_b * hyp_b, axis=-1, keepdims=True)
    xy = g * lax.dot_general(
        m0, hyp_b, (((1,), (1,)), ((), ())),
        preferred_element_type=jnp.float32,
    )
    num = ((1.0 + 2.0 * xy + y2) * g) * m0 + (1.0 - x2) * hyp_b
    denom = 1.0 + 2.0 * xy + x2 * y2
    h = num / jnp.clip(denom, _MIN_NORM, None)
    hn = _rnorm(h)
    pn = jnp.minimum(hn, _MAXNORM)
    f = _artanh(pn) / pn * jnp.where(hn > _MAXNORM, _MAXNORM / hn, 1.0)
    out_ref[...] = f * h


def _stage_pre(x, w, b2d, blk):
    n, d = x.shape
    grid = n // blk
    return pl.pallas_call(
        _pre_body,
        grid=(grid,),
        in_specs=[
            pl.BlockSpec((blk, d), lambda i: (i, 0)),
            pl.BlockSpec((d, d), lambda i: (0, 0)),
            pl.BlockSpec((1, d), lambda i: (0, 0)),
        ],
        out_specs=pl.BlockSpec((blk, d), lambda i: (i, 0)),
        out_shape=jax.ShapeDtypeStruct((n, d), jnp.float32),
    )(x, w, b2d)


def _make_seg_sum(n_acc, d, rows_per_w):
    acc_per_s = n_acc // _NS
    n_pairs = rows_per_w // 2
    tail = rows_per_w - 2 * n_pairs
    if rows_per_w % 8 == 0:
        buf_rows = rows_per_w
    else:
        buf_rows = ((rows_per_w + 14) // 8) * 8
    mesh = plsc.VectorSubcoreMesh(core_axis_name="c", subcore_axis_name="s")

    @functools.partial(
        pl.kernel,
        out_type=jax.ShapeDtypeStruct((_NC, n_acc, d), jnp.float32),
        mesh=mesh,
        scratch_types=[
            pltpu.VMEM((buf_rows, _ECHUNK), jnp.int32),
            pltpu.VMEM((_ECHUNK,), jnp.int32),
            pltpu.VMEM((_ECHUNK,), jnp.int32),
            pltpu.VMEM((_ECHUNK, d), jnp.float32),
            pltpu.VMEM((_ECHUNK, d), jnp.float32),
            pltpu.VMEM_SHARED((n_acc, d), jnp.float32),
            pltpu.SemaphoreType.DMA,
            pltpu.SemaphoreType.DMA,
            pltpu.SemaphoreType.DMA,
            pltpu.SemaphoreType.DMA,
            pltpu.SemaphoreType.DMA,
        ],
    )
    def seg_sum(xt_hbm, src_hbm, dst_hbm, zeros_hbm, out_hbm,
                src_v, dst_v0, dst_v1, rows0, rows1, acc,
                isem, dsem0, dsem1, gsem0, gsem1):
        cid = lax.axis_index("c")
        sid = lax.axis_index("s")
        wid = cid * _NS + sid
        base = rows_per_w * wid
        abase = (base // 8) * 8
        off = base - abase
        icp = pltpu.async_copy(
            src_hbm.at[pl.ds(abase, buf_rows)], src_v, isem)
        pltpu.sync_copy(zeros_hbm, acc.at[pl.ds(sid * acc_per_s, acc_per_s)])
        icp.wait()
        plsc.subcore_barrier()

        if n_pairs:
            pltpu.async_copy(dst_hbm.at[base], dst_v0, dsem0)
            pltpu.async_copy(xt_hbm.at[src_v.at[off]], rows0, gsem0)

        def body(j, carry):
            i = 2 * j
            pltpu.async_copy(dst_hbm.at[base + i + 1], dst_v1, dsem1)
            pltpu.async_copy(xt_hbm.at[src_v.at[off + i + 1]], rows1, gsem1)
            pltpu.make_async_copy(xt_hbm.at[src_v.at[off + i]], rows0, gsem0).wait()
            pltpu.make_async_copy(dst_hbm.at[base + i], dst_v0, dsem0).wait()
            pltpu.sync_copy(rows0, acc.at[dst_v0], add=True)

            @pl.when(j + 1 < n_pairs)
            def _():
                pltpu.async_copy(dst_hbm.at[base + i + 2], dst_v0, dsem0)
                pltpu.async_copy(xt_hbm.at[src_v.at[off + i + 2]], rows0, gsem0)

            pltpu.make_async_copy(
                xt_hbm.at[src_v.at[off + i + 1]], rows1, gsem1).wait()
            pltpu.make_async_copy(
                dst_hbm.at[base + i + 1], dst_v1, dsem1).wait()
            pltpu.sync_copy(rows1, acc.at[dst_v1], add=True)
            return carry

        lax.fori_loop(0, n_pairs, body, 0)
        if tail:
            i_t = 2 * n_pairs
            pltpu.async_copy(dst_hbm.at[base + i_t], dst_v0, dsem0)
            pltpu.async_copy(xt_hbm.at[src_v.at[off + i_t]], rows0, gsem0)
            pltpu.make_async_copy(
                xt_hbm.at[src_v.at[off + i_t]], rows0, gsem0).wait()
            pltpu.make_async_copy(dst_hbm.at[base + i_t], dst_v0, dsem0).wait()
            pltpu.sync_copy(rows0, acc.at[dst_v0], add=True)
        plsc.subcore_barrier()
        pltpu.sync_copy(
            acc.at[pl.ds(sid * acc_per_s, acc_per_s)],
            out_hbm.at[cid, pl.ds(sid * acc_per_s, acc_per_s)],
        )

    return seg_sum


def _post_body(p_ref, out_ref):
    s = p_ref[0] + p_ref[1]
    rs = jax.nn.relu(s)
    sn = _rnorm(s)
    rn = jnp.sqrt(jnp.sum(rs * rs, axis=-1, keepdims=True))
    kappa = jnp.minimum(sn, _ATANH_MAXNORM) / sn
    xn = jnp.clip(kappa * rn, _MIN_NORM, None)
    a2 = jnp.minimum(jnp.tanh(xn), _MAXNORM)
    out_ref[...] = (a2 / xn * kappa) * rs


def _stage_post(partials, n, blk):
    nc, n_acc, d = partials.shape
    grid = n // blk
    return pl.pallas_call(
        _post_body,
        grid=(grid,),
        in_specs=[pl.BlockSpec((nc, blk, d), lambda i: (0, i, 0))],
        out_specs=pl.BlockSpec((blk, d), lambda i: (i, 0)),
        out_shape=jax.ShapeDtypeStruct((n, d), jnp.float32),
    )(partials)


def kernel(x, edge_index, W, b):
    n, d = x.shape
    e = edge_index.shape[1]
    src = edge_index[0]
    dst = edge_index[1]

    grain = _NW * _ECHUNK
    e_pad = ((e + grain - 1) // grain) * grain
    rows_per_w = e_pad // (_NW * _ECHUNK)
    e_buf = e_pad if rows_per_w % 8 == 0 else e_pad + 8 * _ECHUNK
    _ALIGN = _NS * 8
    n_acc = ((n + _ALIGN - 1) // _ALIGN) * _ALIGN
    if e_buf != e:
        pad = e_buf - e
        ar = jnp.arange(pad, dtype=jnp.int32)
        pad_src = ar % n
        pad_dst = n_acc + (ar % 128)
        n_acc += 128
        src = jnp.concatenate([src, pad_src])
        dst = jnp.concatenate([dst, pad_dst])
    src2 = src.reshape(e_buf // _ECHUNK, _ECHUNK)
    dst2 = dst.reshape(e_buf // _ECHUNK, _ECHUNK)

    x_t = _stage_pre(x, W, b.reshape(1, -1), blk=2000)

    zeros = jnp.zeros((n_acc // _NS, d), jnp.float32)
    seg = _make_seg_sum(n_acc, d, rows_per_w)
    partials = seg(x_t, src2, dst2, zeros)

    return _stage_post(partials, n, blk=2000)
